# baseline (speedup 1.0000x reference)
import functools

import jax
import jax.numpy as jnp
from jax import lax
from jax.experimental import pallas as pl
from jax.experimental.pallas import tpu as pltpu, tpu_sc as plsc

N = 10000
E = 320000
NFEAT = 128
HID = 128
NCLASS = 64

NC = 2
NS = 16
NW = NC * NS
B = 80
K = 2
GC = 26
CH = 5
EPT = CH * GC * B
E_PAD = EPT * NW
N_PAD = 10240
RPT = N_PAD // NS

_LEAKY = 0.01


def _make_sc_aggregate(D):
    mesh = plsc.VectorSubcoreMesh(core_axis_name="c", subcore_axis_name="s")
    fvecs = D // 16

    @functools.partial(
        pl.kernel,
        out_type=jax.ShapeDtypeStruct((NC, N_PAD, D), jnp.float32),
        mesh=mesh,
        scratch_types=[
            pltpu.VMEM((GC, B), jnp.int32),
            pltpu.VMEM((GC, B), jnp.int32),
            pltpu.VMEM((GC * B,), jnp.float32),
            pltpu.VMEM((K, B, D), jnp.float32),
            pltpu.VMEM_SHARED((N_PAD, D), jnp.float32),
            pltpu.SemaphoreType.DMA,
            pltpu.SemaphoreType.DMA,
        ],
    )
    def k(support_hbm, src_hbm, dst_hbm, w_hbm, out_hbm,
          src_v, dst_v, w_v, rows_v, acc, g0, g1):
        cid = lax.axis_index("c")
        sid = lax.axis_index("s")
        wid = cid * NS + sid
        gsem = (g0, g1)

        zero = jnp.zeros((16,), jnp.float32)

        def zrow(i, _):
            for f in range(fvecs):
                rows_v[0, i, pl.ds(f * 16, 16)] = zero
            return 0

        lax.fori_loop(0, B, zrow, 0)

        def zcopy(j, _):
            pltpu.sync_copy(rows_v.at[0, pl.ds(0, 64)],
                            acc.at[pl.ds(sid * RPT + j * 64, 64)])
            return 0

        lax.fori_loop(0, RPT // 64, zcopy, 0)
        plsc.subcore_barrier()

        def chunk(c, _):
            pltpu.sync_copy(src_hbm.at[wid, c], src_v)
            pltpu.sync_copy(dst_hbm.at[wid, c], dst_v)
            pltpu.sync_copy(w_hbm.at[wid, c, 0], w_v)

            def scale(b, g):
                def sub(t, _):
                    w16 = w_v[pl.ds(g * B + t * 16, 16)]
                    for k in range(16):
                        wv = w16[k]
                        e = t * 16 + k
                        for f in range(fvecs):
                            rows_v[b, e, pl.ds(f * 16, 16)] = (
                                rows_v[b, e, pl.ds(f * 16, 16)] * wv)
                    return 0

                lax.fori_loop(0, B // 16, sub, 0)

            def gather(b, g):
                pltpu.async_copy(support_hbm.at[src_v.at[g]],
                                 rows_v.at[b], gsem[b])

            def gather_wait(b, g):
                pltpu.make_async_copy(support_hbm.at[src_v.at[g]],
                                      rows_v.at[b], gsem[b]).wait()

            for b in range(2):
                gather(b, b)

            def step(j, _):
                for b in range(2):
                    g = 2 * j + b
                    gather_wait(b, g)
                    scale(b, g)
                    pltpu.sync_copy(rows_v.at[b], acc.at[dst_v.at[g]],
                                    add=True)
                    gather(b, g + 2)
                return 0

            lax.fori_loop(0, GC // 2 - 1, step, 0)

            for b in range(2):
                g = GC - 2 + b
                gather_wait(b, g)
                scale(b, g)
                pltpu.sync_copy(rows_v.at[b], acc.at[dst_v.at[g]], add=True)
            return 0

        lax.fori_loop(0, CH, chunk, 0)
        plsc.subcore_barrier()

        pltpu.sync_copy(acc.at[pl.ds(sid * RPT, RPT)],
                        out_hbm.at[cid, pl.ds(sid * RPT, RPT)])

    return k


_sc_aggregate = _make_sc_aggregate(HID)


_BLK = 1000


def _mm_body(x_ref, w_ref, b_ref, o_ref):
    o_ref[...] = jnp.dot(x_ref[...], w_ref[...],
                         preferred_element_type=jnp.float32) + b_ref[...]


def _first_matmul(x, W, b):
    Din, Dout = W.shape
    return pl.pallas_call(
        _mm_body,
        grid=(N // _BLK,),
        in_specs=[
            pl.BlockSpec((_BLK, Din), lambda i: (i, 0)),
            pl.BlockSpec((Din, Dout), lambda i: (0, 0)),
            pl.BlockSpec((1, Dout), lambda i: (0, 0)),
        ],
        out_specs=pl.BlockSpec((_BLK, Dout), lambda i: (i, 0)),
        out_shape=jax.ShapeDtypeStruct((N, Dout), jnp.float32),
    )(x, W, b.reshape(1, Dout))


def _fused_body(p0_ref, p1_ref, w_ref, b_ref, o_ref):
    h = p0_ref[0] + p1_ref[0]
    h = jnp.where(h >= 0, h, _LEAKY * h)
    o_ref[...] = jnp.dot(h, w_ref[...],
                         preferred_element_type=jnp.float32) + b_ref[...]


def _fused_matmul(p, W, b):
    Din, Dout = W.shape
    return pl.pallas_call(
        _fused_body,
        grid=(N // _BLK,),
        in_specs=[
            pl.BlockSpec((1, _BLK, Din), lambda i: (0, i, 0)),
            pl.BlockSpec((1, _BLK, Din), lambda i: (1, i, 0)),
            pl.BlockSpec((Din, Dout), lambda i: (0, 0)),
            pl.BlockSpec((1, Dout), lambda i: (0, 0)),
        ],
        out_specs=pl.BlockSpec((_BLK, Dout), lambda i: (i, 0)),
        out_shape=jax.ShapeDtypeStruct((N, Dout), jnp.float32),
    )(p, p, W, b.reshape(1, Dout))


def _final_body(p0_ref, p1_ref, o_ref):
    h = p0_ref[0] + p1_ref[0]
    o_ref[...] = jnp.where(h >= 0, h, _LEAKY * h)[:, :NCLASS]


def _final_act(p):
    D = p.shape[2]
    return pl.pallas_call(
        _final_body,
        grid=(N // _BLK,),
        in_specs=[
            pl.BlockSpec((1, _BLK, D), lambda i: (0, i, 0)),
            pl.BlockSpec((1, _BLK, D), lambda i: (1, i, 0)),
        ],
        out_specs=pl.BlockSpec((_BLK, NCLASS), lambda i: (i, 0)),
        out_shape=jax.ShapeDtypeStruct((N, NCLASS), jnp.float32),
    )(p, p)


def kernel(x, edge_index, edge_weight, W1, b1, W2, b2, W3, b3):
    npad = E_PAD - E
    src = jnp.pad(edge_index[0].astype(jnp.int32), (0, npad)).reshape(NW, CH, GC, B)
    dst = jnp.pad(edge_index[1].astype(jnp.int32), (0, npad)).reshape(NW, CH, GC, B)
    w = jnp.pad(edge_weight.astype(jnp.float32), (0, npad)).reshape(NW, CH, 1, GC * B)

    W3p = jnp.pad(W3, ((0, 0), (0, HID - NCLASS)))
    b3p = jnp.pad(b3, (0, HID - NCLASS))

    s = _first_matmul(x, W1, b1)
    p = _sc_aggregate(s, src, dst, w)
    s = _fused_matmul(p, W2, b2)
    p = _sc_aggregate(s, src, dst, w)
    s = _fused_matmul(p, W3p, b3p)
    p = _sc_aggregate(s, src, dst, w)
    return _final_act(p)


# speedup vs baseline: 1.5573x; 1.5573x over previous
import functools

import jax
import jax.numpy as jnp
from jax import lax
from jax.experimental import pallas as pl
from jax.experimental.pallas import tpu as pltpu, tpu_sc as plsc

N = 10000
E = 320000
NFEAT = 128
HID = 128
NCLASS = 64

NC = 2
NS = 16
NW = NC * NS
B = 96
K = 1
GC = 21
CH = 5
EPT = CH * GC * B
E_PAD = EPT * NW
N_PAD = 10240
RPT = N_PAD // NS

_LEAKY = 0.01


def _make_sc_aggregate(D):
    mesh = plsc.VectorSubcoreMesh(core_axis_name="c", subcore_axis_name="s")
    fvecs = D // 16

    @functools.partial(
        pl.kernel,
        out_type=jax.ShapeDtypeStruct((NC, N_PAD, D), jnp.float32),
        mesh=mesh,
        scratch_types=[
            pltpu.VMEM((GC, B), jnp.int32),
            pltpu.VMEM((GC, B), jnp.int32),
            pltpu.VMEM((GC * B,), jnp.float32),
            pltpu.VMEM((K, B, D), jnp.float32),
            pltpu.VMEM_SHARED((N_PAD, D), jnp.float32),
            pltpu.SemaphoreType.DMA,
            pltpu.SemaphoreType.DMA,
        ],
    )
    def k(support_hbm, src_hbm, dst_hbm, w_hbm, out_hbm,
          src_v, dst_v, w_v, rows_v, acc, g0, g1):
        cid = lax.axis_index("c")
        sid = lax.axis_index("s")
        wid = cid * NS + sid
        gsem = (g0, g1)

        zero = jnp.zeros((16,), jnp.float32)

        def zrow(i, _):
            for f in range(fvecs):
                rows_v[0, i, pl.ds(f * 16, 16)] = zero
            return 0

        lax.fori_loop(0, B, zrow, 0)

        def zcopy(j, _):
            pltpu.sync_copy(rows_v.at[0, pl.ds(0, 64)],
                            acc.at[pl.ds(sid * RPT + j * 64, 64)])
            return 0

        lax.fori_loop(0, RPT // 64, zcopy, 0)
        plsc.subcore_barrier()

        def chunk(c, _):
            pltpu.sync_copy(src_hbm.at[wid, c], src_v)
            pltpu.sync_copy(dst_hbm.at[wid, c], dst_v)
            pltpu.sync_copy(w_hbm.at[wid, c, 0], w_v)

            def scale(b, g):
                def sub(t, _):
                    w16 = w_v[pl.ds(g * B + t * 16, 16)]
                    for k in range(16):
                        wv = w16[k]
                        e = t * 16 + k
                        for f in range(fvecs):
                            rows_v[b, e, pl.ds(f * 16, 16)] = (
                                rows_v[b, e, pl.ds(f * 16, 16)] * wv)
                    return 0

                lax.fori_loop(0, B // 16, sub, 0)

            def grp(g, _):
                pltpu.async_copy(support_hbm.at[src_v.at[g]],
                                 rows_v.at[0], g0).wait()
                scale(0, g)
                pltpu.sync_copy(rows_v.at[0], acc.at[dst_v.at[g]], add=True)
                return 0

            lax.fori_loop(0, GC, grp, 0)
            return 0

        lax.fori_loop(0, CH, chunk, 0)
        plsc.subcore_barrier()

        pltpu.sync_copy(acc.at[pl.ds(sid * RPT, RPT)],
                        out_hbm.at[cid, pl.ds(sid * RPT, RPT)])

    return k


_sc_aggregate = _make_sc_aggregate(HID)


_BLK = 1000


def _mm_body(x_ref, w_ref, b_ref, o_ref):
    o_ref[...] = jnp.dot(x_ref[...], w_ref[...],
                         preferred_element_type=jnp.float32) + b_ref[...]


def _first_matmul(x, W, b):
    Din, Dout = W.shape
    return pl.pallas_call(
        _mm_body,
        grid=(N // _BLK,),
        in_specs=[
            pl.BlockSpec((_BLK, Din), lambda i: (i, 0)),
            pl.BlockSpec((Din, Dout), lambda i: (0, 0)),
            pl.BlockSpec((1, Dout), lambda i: (0, 0)),
        ],
        out_specs=pl.BlockSpec((_BLK, Dout), lambda i: (i, 0)),
        out_shape=jax.ShapeDtypeStruct((N, Dout), jnp.float32),
    )(x, W, b.reshape(1, Dout))


def _fused_body(p0_ref, p1_ref, w_ref, b_ref, o_ref):
    h = p0_ref[0] + p1_ref[0]
    h = jnp.where(h >= 0, h, _LEAKY * h)
    o_ref[...] = jnp.dot(h, w_ref[...],
                         preferred_element_type=jnp.float32) + b_ref[...]


def _fused_matmul(p, W, b):
    Din, Dout = W.shape
    return pl.pallas_call(
        _fused_body,
        grid=(N // _BLK,),
        in_specs=[
            pl.BlockSpec((1, _BLK, Din), lambda i: (0, i, 0)),
            pl.BlockSpec((1, _BLK, Din), lambda i: (1, i, 0)),
            pl.BlockSpec((Din, Dout), lambda i: (0, 0)),
            pl.BlockSpec((1, Dout), lambda i: (0, 0)),
        ],
        out_specs=pl.BlockSpec((_BLK, Dout), lambda i: (i, 0)),
        out_shape=jax.ShapeDtypeStruct((N, Dout), jnp.float32),
    )(p, p, W, b.reshape(1, Dout))


def _final_body(p0_ref, p1_ref, o_ref):
    h = p0_ref[0] + p1_ref[0]
    o_ref[...] = jnp.where(h >= 0, h, _LEAKY * h)[:, :NCLASS]


def _final_act(p):
    D = p.shape[2]
    return pl.pallas_call(
        _final_body,
        grid=(N // _BLK,),
        in_specs=[
            pl.BlockSpec((1, _BLK, D), lambda i: (0, i, 0)),
            pl.BlockSpec((1, _BLK, D), lambda i: (1, i, 0)),
        ],
        out_specs=pl.BlockSpec((_BLK, NCLASS), lambda i: (i, 0)),
        out_shape=jax.ShapeDtypeStruct((N, NCLASS), jnp.float32),
    )(p, p)


def kernel(x, edge_index, edge_weight, W1, b1, W2, b2, W3, b3):
    npad = E_PAD - E
    src = jnp.pad(edge_index[0].astype(jnp.int32), (0, npad)).reshape(NW, CH, GC, B)
    dst = jnp.pad(edge_index[1].astype(jnp.int32), (0, npad)).reshape(NW, CH, GC, B)
    w = jnp.pad(edge_weight.astype(jnp.float32), (0, npad)).reshape(NW, CH, 1, GC * B)

    W3p = jnp.pad(W3, ((0, 0), (0, HID - NCLASS)))
    b3p = jnp.pad(b3, (0, HID - NCLASS))

    s = _first_matmul(x, W1, b1)
    p = _sc_aggregate(s, src, dst, w)
    s = _fused_matmul(p, W2, b2)
    p = _sc_aggregate(s, src, dst, w)
    s = _fused_matmul(p, W3p, b3p)
    p = _sc_aggregate(s, src, dst, w)
    return _final_act(p)


# speedup vs baseline: 2.2692x; 1.4571x over previous
import functools

import jax
import jax.numpy as jnp
from jax import lax
from jax.experimental import pallas as pl
from jax.experimental.pallas import tpu as pltpu, tpu_sc as plsc

N = 10000
E = 320000
NFEAT = 128
HID = 128
NCLASS = 64

NC = 2
NS = 16
NW = NC * NS
B = 80
K = 1
GC = 25
CH = 5
EPT = CH * GC * B
E_PAD = EPT * NW
N_PAD = 10240
RPT = N_PAD // NS

_LEAKY = 0.01


def _make_sc_aggregate(D):
    mesh = plsc.VectorSubcoreMesh(core_axis_name="c", subcore_axis_name="s")
    fvecs = D // 16

    @functools.partial(
        pl.kernel,
        out_type=jax.ShapeDtypeStruct((NC, N_PAD, D), jnp.float32),
        mesh=mesh,
        scratch_types=[
            pltpu.VMEM((2, GC, B), jnp.int32),
            pltpu.VMEM((GC * B,), jnp.float32),
            pltpu.VMEM((K, B, D), jnp.float32),
            pltpu.VMEM_SHARED((N_PAD, D), jnp.float32),
            pltpu.SemaphoreType.DMA,
        ],
    )
    def k(support_hbm, edges_hbm, w_hbm, out_hbm, ed_v, w_v, rows_v, acc, g0):
        cid = lax.axis_index("c")
        sid = lax.axis_index("s")
        wid = cid * NS + sid

        zero = jnp.zeros((16,), jnp.float32)

        def zrow(i, _):
            for f in range(fvecs):
                rows_v[0, i, pl.ds(f * 16, 16)] = zero
            return 0

        lax.fori_loop(0, B, zrow, 0)

        def zcopy(j, _):
            pltpu.sync_copy(rows_v.at[0, pl.ds(0, 64)],
                            acc.at[pl.ds(sid * RPT + j * 64, 64)])
            return 0

        lax.fori_loop(0, RPT // 64, zcopy, 0)
        plsc.subcore_barrier()

        def chunk(c, _):
            pltpu.sync_copy(edges_hbm.at[wid, c], ed_v)
            pltpu.sync_copy(w_hbm.at[wid, c, 0], w_v)

            def scale(b, g):
                def sub(t, _):
                    w16 = w_v[pl.ds(g * B + t * 16, 16)]
                    for k in range(16):
                        wv = w16[k]
                        e = t * 16 + k
                        for f in range(fvecs):
                            rows_v[b, e, pl.ds(f * 16, 16)] = (
                                rows_v[b, e, pl.ds(f * 16, 16)] * wv)
                    return 0

                lax.fori_loop(0, B // 16, sub, 0)

            def grp(g, _):
                pltpu.async_copy(support_hbm.at[ed_v.at[0, g]],
                                 rows_v.at[0], g0).wait()
                scale(0, g)
                pltpu.sync_copy(rows_v.at[0], acc.at[ed_v.at[1, g]], add=True)
                return 0

            lax.fori_loop(0, GC, grp, 0)
            return 0

        lax.fori_loop(0, CH, chunk, 0)
        plsc.subcore_barrier()

        pltpu.sync_copy(acc.at[pl.ds(sid * RPT, RPT)],
                        out_hbm.at[cid, pl.ds(sid * RPT, RPT)])

    return k


_sc_aggregate = _make_sc_aggregate(HID)


_BLK = 1000


def _mm_body(x_ref, w_ref, b_ref, o_ref):
    o_ref[...] = jnp.dot(x_ref[...], w_ref[...],
                         preferred_element_type=jnp.float32) + b_ref[...]


def _first_matmul(x, W, b):
    Din, Dout = W.shape
    return pl.pallas_call(
        _mm_body,
        grid=(N // _BLK,),
        in_specs=[
            pl.BlockSpec((_BLK, Din), lambda i: (i, 0)),
            pl.BlockSpec((Din, Dout), lambda i: (0, 0)),
            pl.BlockSpec((1, Dout), lambda i: (0, 0)),
        ],
        out_specs=pl.BlockSpec((_BLK, Dout), lambda i: (i, 0)),
        out_shape=jax.ShapeDtypeStruct((N, Dout), jnp.float32),
    )(x, W, b.reshape(1, Dout))


def _fused_body(p0_ref, p1_ref, w_ref, b_ref, o_ref):
    h = p0_ref[0] + p1_ref[0]
    h = jnp.where(h >= 0, h, _LEAKY * h)
    o_ref[...] = jnp.dot(h, w_ref[...],
                         preferred_element_type=jnp.float32) + b_ref[...]


def _fused_matmul(p, W, b):
    Din, Dout = W.shape
    return pl.pallas_call(
        _fused_body,
        grid=(N // _BLK,),
        in_specs=[
            pl.BlockSpec((1, _BLK, Din), lambda i: (0, i, 0)),
            pl.BlockSpec((1, _BLK, Din), lambda i: (1, i, 0)),
            pl.BlockSpec((Din, Dout), lambda i: (0, 0)),
            pl.BlockSpec((1, Dout), lambda i: (0, 0)),
        ],
        out_specs=pl.BlockSpec((_BLK, Dout), lambda i: (i, 0)),
        out_shape=jax.ShapeDtypeStruct((N, Dout), jnp.float32),
    )(p, p, W, b.reshape(1, Dout))


def _final_body(p0_ref, p1_ref, o_ref):
    h = p0_ref[0] + p1_ref[0]
    o_ref[...] = jnp.where(h >= 0, h, _LEAKY * h)[:, :NCLASS]


def _final_act(p):
    D = p.shape[2]
    return pl.pallas_call(
        _final_body,
        grid=(N // _BLK,),
        in_specs=[
            pl.BlockSpec((1, _BLK, D), lambda i: (0, i, 0)),
            pl.BlockSpec((1, _BLK, D), lambda i: (1, i, 0)),
        ],
        out_specs=pl.BlockSpec((_BLK, NCLASS), lambda i: (i, 0)),
        out_shape=jax.ShapeDtypeStruct((N, NCLASS), jnp.float32),
    )(p, p)


def kernel(x, edge_index, edge_weight, W1, b1, W2, b2, W3, b3):
    npad = E_PAD - E
    src = jnp.pad(edge_index[0].astype(jnp.int32), (0, npad)).reshape(NW, CH, GC, B)
    dst = jnp.pad(edge_index[1].astype(jnp.int32), (0, npad)).reshape(NW, CH, GC, B)
    edges = jnp.stack([src, dst], axis=2)
    w = jnp.pad(edge_weight.astype(jnp.float32), (0, npad)).reshape(NW, CH, 1, GC * B)

    W3p = jnp.pad(W3, ((0, 0), (0, HID - NCLASS)))
    b3p = jnp.pad(b3, (0, HID - NCLASS))

    s = _first_matmul(x, W1, b1)
    p = _sc_aggregate(s, edges, w)
    s = _fused_matmul(p, W2, b2)
    p = _sc_aggregate(s, edges, w)
    s = _fused_matmul(p, W3p, b3p)
    p = _sc_aggregate(s, edges, w)
    return _final_act(p)
